# baseline (speedup 1.0000x reference)
import functools

import jax
import jax.numpy as jnp
from jax import lax
from jax.experimental import pallas as pl
from jax.experimental.pallas import tpu as pltpu
from jax.experimental.pallas import tpu_sc as plsc

N = 10000
NPAD = 10240
D = 128
E = 320000
NC = 2
NS = 16
NW = NC * NS
B = 128
GSTEPS = 80
EPAD = NW * GSTEPS * B
RPT = NPAD // NS
GCH = 16
BLK = 512


def _pre_body(x_ref, msg_w_ref, skip_w_ref, skip_b_ref, msg_b_ref, y_ref, skip_ref):
    xb = x_ref[...]
    dn = (((1,), (1,)), ((), ()))
    y_ref[...] = lax.dot_general(xb, msg_w_ref[...], dn,
                                 preferred_element_type=jnp.float32)
    skip_ref[...] = (lax.dot_general(xb, skip_w_ref[...], dn,
                                     preferred_element_type=jnp.float32)
                     + skip_b_ref[...] + msg_b_ref[...])


def _pre(x_pad, msg_w, skip_w, skip_b, msg_b):
    grid = (NPAD // BLK,)
    return pl.pallas_call(
        _pre_body,
        grid=grid,
        in_specs=[
            pl.BlockSpec((BLK, D), lambda i: (i, 0)),
            pl.BlockSpec((D, D), lambda i: (0, 0)),
            pl.BlockSpec((D, D), lambda i: (0, 0)),
            pl.BlockSpec((1, D), lambda i: (0, 0)),
            pl.BlockSpec((1, D), lambda i: (0, 0)),
        ],
        out_specs=[
            pl.BlockSpec((BLK, D), lambda i: (i, 0)),
            pl.BlockSpec((BLK, D), lambda i: (i, 0)),
        ],
        out_shape=[
            jax.ShapeDtypeStruct((NPAD, D), jnp.float32),
            jax.ShapeDtypeStruct((NPAD, D), jnp.float32),
        ],
    )(x_pad, msg_w, skip_w, skip_b, msg_b)


def _sc_body(srcd_hbm, srcg_hbm, dst_hbm, y_hbm,
             dis_hbm, feat_hbm, agg_hbm,
             idx_a, idx_b, onesv, zerosv, degv, disv,
             fbuf, gbuf0, gbuf1, gbuf2, deg_sh, agg_sh, sem, sem2, sem3):
    cid = lax.axis_index("c")
    sid = lax.axis_index("s")
    w = cid * NS + sid
    base = sid * RPT

    fzero16 = jnp.zeros((16,), jnp.float32)
    fone16 = jnp.ones((16,), jnp.float32)

    def _initz(i, c):
        zerosv[pl.ds(i * 16, 16)] = fzero16
        return c
    lax.fori_loop(0, RPT // 16, _initz, 0)
    for j in range(B // 16):
        onesv[pl.ds(j * 16, 16)] = fone16

    def _zrow(i, c):
        for j in range(D // 16):
            fbuf[i, pl.ds(j * 16, 16)] = fzero16
        return c
    lax.fori_loop(0, B, _zrow, 0)

    pltpu.sync_copy(zerosv, deg_sh.at[pl.ds(base, RPT)])
    for q in range(RPT // B):
        pltpu.sync_copy(fbuf, agg_sh.at[pl.ds(base + q * B, B)])
    plsc.subcore_barrier()

    for k in range(2):
        for q in range(GSTEPS // GCH):
            pltpu.sync_copy(srcd_hbm.at[2 * sid + k, pl.ds(q * GCH, GCH)], idx_a)

            def _dadd(g, c):
                pltpu.async_copy(onesv, deg_sh.at[idx_a.at[g]], sem, add=True)
                return c
            lax.fori_loop(0, GCH, _dadd, 0)

            def _ddrain(g, c):
                pltpu.make_async_copy(y_hbm.at[0], onesv, sem).wait()
                return c
            lax.fori_loop(0, GCH, _ddrain, 0)
    plsc.subcore_barrier()

    pltpu.sync_copy(deg_sh.at[pl.ds(base, RPT)], degv)

    def _rsq(i, c):
        d = degv[pl.ds(i * 16, 16)]
        bits = lax.bitcast_convert_type(d, jnp.int32)
        y0 = lax.bitcast_convert_type(
            jnp.int32(0x5F3759DF) - lax.shift_right_logical(bits, 1), jnp.float32)
        nh = d * -0.5
        y0 = y0 * (1.5 + nh * y0 * y0)
        y0 = y0 * (1.5 + nh * y0 * y0)
        y0 = y0 * (1.5 + nh * y0 * y0)
        disv[pl.ds(i * 16, 16)] = jnp.where(d > 0.5, y0, 0.0)
        return c
    lax.fori_loop(0, RPT // 16, _rsq, 0)
    for j in range(RPT // B):
        pltpu.sync_copy(disv.at[pl.ds(j * B, B)],
                        dis_hbm.at[cid, sid * (RPT // B) + j])

    for q in range(RPT // B):
        rbase = base + q * B
        pltpu.sync_copy(y_hbm.at[pl.ds(rbase, B)], fbuf)

        def _scale(t, c, _q=q):
            dv = disv[pl.ds(_q * B + t * 16, 16)]
            for k in range(16):
                r = t * 16 + k
                dsc = dv[k]
                for j in range(D // 32):
                    a = fbuf[r, pl.ds(j * 32, 16)] * dsc
                    b = fbuf[r, pl.ds(j * 32 + 16, 16)] * dsc
                    gbuf0[r, pl.ds(j * 32, 32)] = plsc.pack(
                        a, b, format=plsc.PackFormat.INTERLEAVED)
            return c
        lax.fori_loop(0, B // 16, _scale, 0)
        pltpu.sync_copy(gbuf0, feat_hbm.at[pl.ds(cid * NPAD + rbase, B)])
    plsc.subcore_barrier()

    def _unpack_scatter(gb, g):
        def _cv(r, c):
            for j in range(D // 32):
                ab = gb[r, pl.ds(j * 32, 32)]
                a, b = plsc.unpack(ab, format=plsc.PackFormat.INTERLEAVED)
                fbuf[r, pl.ds(j * 32, 16)] = a
                fbuf[r, pl.ds(j * 32 + 16, 16)] = b
            return c
        lax.fori_loop(0, B, _cv, 0)
        pltpu.sync_copy(fbuf, agg_sh.at[idx_b.at[g]], add=True)

    gbufs = (gbuf0, gbuf1, gbuf2)
    gsems = (sem, sem2, sem3)
    for q in range(GSTEPS // GCH):
        pltpu.sync_copy(srcg_hbm.at[cid, w, pl.ds(q * GCH, GCH)], idx_a)
        pltpu.sync_copy(dst_hbm.at[w, pl.ds(q * GCH, GCH)], idx_b)

        pltpu.async_copy(feat_hbm.at[idx_a.at[0]], gbufs[0], gsems[0])
        pltpu.async_copy(feat_hbm.at[idx_a.at[1]], gbufs[1], gsems[1])
        for g in range(GCH):
            r = g % 3
            pltpu.make_async_copy(feat_hbm.at[pl.ds(0, B)],
                                  gbufs[r], gsems[r]).wait()
            if g + 2 < GCH:
                r2 = (g + 2) % 3
                pltpu.async_copy(feat_hbm.at[idx_a.at[g + 2]],
                                 gbufs[r2], gsems[r2])
            _unpack_scatter(gbufs[r], g)
    plsc.subcore_barrier()

    pltpu.sync_copy(agg_sh.at[pl.ds(base, RPT)],
                    agg_hbm.at[cid, pl.ds(base, RPT)])


def _sc_agg(srcd, srcg, dst, y):
    mesh = plsc.VectorSubcoreMesh(core_axis_name="c", subcore_axis_name="s")
    fn = functools.partial(
        pl.kernel,
        out_type=(
            jax.ShapeDtypeStruct((NC, NPAD // B, B), jnp.float32),
            jax.ShapeDtypeStruct((NC * NPAD, D), jnp.bfloat16),
            jax.ShapeDtypeStruct((NC, NPAD, D), jnp.float32),
        ),
        mesh=mesh,
        compiler_params=pltpu.CompilerParams(use_tc_tiling_on_sc=False,
                                             needs_layout_passes=False),
        scratch_types=[
            pltpu.VMEM((GCH, B), jnp.int32),
            pltpu.VMEM((GCH, B), jnp.int32),
            pltpu.VMEM((B,), jnp.float32),
            pltpu.VMEM((RPT,), jnp.float32),
            pltpu.VMEM((RPT,), jnp.float32),
            pltpu.VMEM((RPT,), jnp.float32),
            pltpu.VMEM((B, D), jnp.float32),
            pltpu.VMEM((B, D), jnp.bfloat16),
            pltpu.VMEM((B, D), jnp.bfloat16),
            pltpu.VMEM((B, D), jnp.bfloat16),
            pltpu.VMEM_SHARED((NPAD,), jnp.float32),
            pltpu.VMEM_SHARED((NPAD, D), jnp.float32),
            pltpu.SemaphoreType.DMA,
            pltpu.SemaphoreType.DMA,
            pltpu.SemaphoreType.DMA,
        ],
    )(_sc_body)
    return fn(srcd, srcg, dst, y)


def _comb_body(agg_ref, dis_ref, skip_ref, out_ref):
    out_ref[...] = ((agg_ref[0] + agg_ref[1]) * dis_ref[...] + skip_ref[...])


def _combine(agg, dis_col, skip):
    grid = (NPAD // BLK,)
    return pl.pallas_call(
        _comb_body,
        grid=grid,
        in_specs=[
            pl.BlockSpec((NC, BLK, D), lambda i: (0, i, 0)),
            pl.BlockSpec((BLK, 1), lambda i: (i, 0)),
            pl.BlockSpec((BLK, D), lambda i: (i, 0)),
        ],
        out_specs=pl.BlockSpec((BLK, D), lambda i: (i, 0)),
        out_shape=jax.ShapeDtypeStruct((NPAD, D), jnp.float32),
    )(agg, dis_col, skip)


def kernel(x, edge_index, skip_w, skip_b, msg_w, msg_b):
    x = x.astype(jnp.float32)
    ei = edge_index.astype(jnp.int32)

    x_pad = jnp.zeros((NPAD, D), jnp.float32).at[:N].set(x)

    pad = jnp.full((2, EPAD - E), NPAD - 1, jnp.int32)
    ei_p = jnp.concatenate([ei, pad], axis=1)
    src = ei_p[0].reshape(NW, GSTEPS, B)
    dst = ei_p[1].reshape(NW, GSTEPS, B)
    srcg = jnp.stack([src, src + NPAD], axis=0)

    y, skip = _pre(x_pad, msg_w, skip_w,
                   skip_b.reshape(1, D), msg_b.reshape(1, D))
    dis, _feat, agg = _sc_agg(src, srcg, dst, y)
    out = _combine(agg, dis[0].reshape(NPAD, 1), skip)
    return out[:N]


# speedup vs baseline: 1.0430x; 1.0430x over previous
import functools

import jax
import jax.numpy as jnp
from jax import lax
from jax.experimental import pallas as pl
from jax.experimental.pallas import tpu as pltpu
from jax.experimental.pallas import tpu_sc as plsc

N = 10000
NPAD = 10240
D = 128
E = 320000
NC = 2
NS = 16
NW = NC * NS
B = 128
GSTEPS = 80
EPAD = NW * GSTEPS * B
RPT = NPAD // NS
GCH = 40
BLK = 512


def _pre_body(x_ref, msg_w_ref, skip_w_ref, skip_b_ref, msg_b_ref, y_ref, skip_ref):
    xb = x_ref[...]
    dn = (((1,), (1,)), ((), ()))
    y_ref[...] = lax.dot_general(xb, msg_w_ref[...], dn,
                                 preferred_element_type=jnp.float32)
    skip_ref[...] = (lax.dot_general(xb, skip_w_ref[...], dn,
                                     preferred_element_type=jnp.float32)
                     + skip_b_ref[...] + msg_b_ref[...])


def _pre(x_pad, msg_w, skip_w, skip_b, msg_b):
    grid = (NPAD // BLK,)
    return pl.pallas_call(
        _pre_body,
        grid=grid,
        in_specs=[
            pl.BlockSpec((BLK, D), lambda i: (i, 0)),
            pl.BlockSpec((D, D), lambda i: (0, 0)),
            pl.BlockSpec((D, D), lambda i: (0, 0)),
            pl.BlockSpec((1, D), lambda i: (0, 0)),
            pl.BlockSpec((1, D), lambda i: (0, 0)),
        ],
        out_specs=[
            pl.BlockSpec((BLK, D), lambda i: (i, 0)),
            pl.BlockSpec((BLK, D), lambda i: (i, 0)),
        ],
        out_shape=[
            jax.ShapeDtypeStruct((NPAD, D), jnp.float32),
            jax.ShapeDtypeStruct((NPAD, D), jnp.float32),
        ],
    )(x_pad, msg_w, skip_w, skip_b, msg_b)


def _sc_body(srcd_hbm, srcg_hbm, dst_hbm, y_hbm,
             dis_hbm, feat_hbm, agg_hbm,
             idx_a, idx_b, onesv, zerosv, degv, disv,
             fbuf, gbuf0, gbuf1, deg_sh, agg_sh, sem, sem2):
    cid = lax.axis_index("c")
    sid = lax.axis_index("s")
    w = cid * NS + sid
    base = sid * RPT

    fzero16 = jnp.zeros((16,), jnp.float32)
    fone16 = jnp.ones((16,), jnp.float32)

    def _initz(i, c):
        zerosv[pl.ds(i * 16, 16)] = fzero16
        return c
    lax.fori_loop(0, RPT // 16, _initz, 0)
    for j in range(B // 16):
        onesv[pl.ds(j * 16, 16)] = fone16

    def _zrow(i, c):
        for j in range(D // 16):
            fbuf[i, pl.ds(j * 16, 16)] = fzero16
        return c
    lax.fori_loop(0, B, _zrow, 0)

    pltpu.sync_copy(zerosv, deg_sh.at[pl.ds(base, RPT)])
    for q in range(RPT // B):
        pltpu.sync_copy(fbuf, agg_sh.at[pl.ds(base + q * B, B)])
    plsc.subcore_barrier()

    for k in range(2):
        for q in range(GSTEPS // GCH):
            pltpu.sync_copy(srcd_hbm.at[2 * sid + k, pl.ds(q * GCH, GCH)], idx_a)

            def _dadd(g, c):
                pltpu.async_copy(onesv, deg_sh.at[idx_a.at[g]], sem, add=True)
                return c
            lax.fori_loop(0, GCH, _dadd, 0)

            def _ddrain(g, c):
                pltpu.make_async_copy(y_hbm.at[0], onesv, sem).wait()
                return c
            lax.fori_loop(0, GCH, _ddrain, 0)
    plsc.subcore_barrier()

    pltpu.sync_copy(deg_sh.at[pl.ds(base, RPT)], degv)

    def _rsq(i, c):
        d = degv[pl.ds(i * 16, 16)]
        bits = lax.bitcast_convert_type(d, jnp.int32)
        y0 = lax.bitcast_convert_type(
            jnp.int32(0x5F3759DF) - lax.shift_right_logical(bits, 1), jnp.float32)
        nh = d * -0.5
        y0 = y0 * (1.5 + nh * y0 * y0)
        y0 = y0 * (1.5 + nh * y0 * y0)
        y0 = y0 * (1.5 + nh * y0 * y0)
        disv[pl.ds(i * 16, 16)] = jnp.where(d > 0.5, y0, 0.0)
        return c
    lax.fori_loop(0, RPT // 16, _rsq, 0)
    for j in range(RPT // B):
        pltpu.sync_copy(disv.at[pl.ds(j * B, B)],
                        dis_hbm.at[cid, sid * (RPT // B) + j])

    for q in range(RPT // B):
        rbase = base + q * B
        pltpu.sync_copy(y_hbm.at[pl.ds(rbase, B)], fbuf)

        def _scale(t, c, _q=q):
            dv = disv[pl.ds(_q * B + t * 16, 16)]
            for k in range(16):
                r = t * 16 + k
                dsc = dv[k]
                for j in range(D // 32):
                    a = fbuf[r, pl.ds(j * 32, 16)] * dsc
                    b = fbuf[r, pl.ds(j * 32 + 16, 16)] * dsc
                    gbuf0[r, pl.ds(j * 32, 32)] = plsc.pack(
                        a, b, format=plsc.PackFormat.INTERLEAVED)
            return c
        lax.fori_loop(0, B // 16, _scale, 0)
        pltpu.sync_copy(gbuf0, feat_hbm.at[pl.ds(cid * NPAD + rbase, B)])
    plsc.subcore_barrier()

    def _unpack_scatter(gb, g):
        def _cv(r, c):
            for j in range(D // 32):
                ab = gb[r, pl.ds(j * 32, 32)]
                a, b = plsc.unpack(ab, format=plsc.PackFormat.INTERLEAVED)
                fbuf[r, pl.ds(j * 32, 16)] = a
                fbuf[r, pl.ds(j * 32 + 16, 16)] = b
            return c
        lax.fori_loop(0, B, _cv, 0)
        pltpu.sync_copy(fbuf, agg_sh.at[idx_b.at[g]], add=True)

    for q in range(GSTEPS // GCH):
        pltpu.sync_copy(srcg_hbm.at[cid, w, pl.ds(q * GCH, GCH)], idx_a)
        pltpu.sync_copy(dst_hbm.at[w, pl.ds(q * GCH, GCH)], idx_b)

        pltpu.async_copy(feat_hbm.at[idx_a.at[0]], gbuf0, sem)

        def _edge2(h, c):
            g0 = 2 * h
            pltpu.async_copy(feat_hbm.at[idx_a.at[g0 + 1]], gbuf1, sem2)
            pltpu.make_async_copy(feat_hbm.at[pl.ds(0, B)], gbuf0, sem).wait()
            _unpack_scatter(gbuf0, g0)

            @pl.when(h < GCH // 2 - 1)
            def _():
                pltpu.async_copy(feat_hbm.at[idx_a.at[g0 + 2]], gbuf0, sem)
            pltpu.make_async_copy(feat_hbm.at[pl.ds(0, B)], gbuf1, sem2).wait()
            _unpack_scatter(gbuf1, g0 + 1)
            return c
        lax.fori_loop(0, GCH // 2, _edge2, 0)
    plsc.subcore_barrier()

    pltpu.sync_copy(agg_sh.at[pl.ds(base, RPT)],
                    agg_hbm.at[cid, pl.ds(base, RPT)])


def _sc_agg(srcd, srcg, dst, y):
    mesh = plsc.VectorSubcoreMesh(core_axis_name="c", subcore_axis_name="s")
    fn = functools.partial(
        pl.kernel,
        out_type=(
            jax.ShapeDtypeStruct((NC, NPAD // B, B), jnp.float32),
            jax.ShapeDtypeStruct((NC * NPAD, D), jnp.bfloat16),
            jax.ShapeDtypeStruct((NC, NPAD, D), jnp.float32),
        ),
        mesh=mesh,
        compiler_params=pltpu.CompilerParams(use_tc_tiling_on_sc=False,
                                             needs_layout_passes=False),
        scratch_types=[
            pltpu.VMEM((GCH, B), jnp.int32),
            pltpu.VMEM((GCH, B), jnp.int32),
            pltpu.VMEM((B,), jnp.float32),
            pltpu.VMEM((RPT,), jnp.float32),
            pltpu.VMEM((RPT,), jnp.float32),
            pltpu.VMEM((RPT,), jnp.float32),
            pltpu.VMEM((B, D), jnp.float32),
            pltpu.VMEM((B, D), jnp.bfloat16),
            pltpu.VMEM((B, D), jnp.bfloat16),
            pltpu.VMEM_SHARED((NPAD,), jnp.float32),
            pltpu.VMEM_SHARED((NPAD, D), jnp.float32),
            pltpu.SemaphoreType.DMA,
            pltpu.SemaphoreType.DMA,
        ],
    )(_sc_body)
    return fn(srcd, srcg, dst, y)


def _comb_body(agg_ref, dis_ref, skip_ref, out_ref):
    out_ref[...] = ((agg_ref[0] + agg_ref[1]) * dis_ref[...] + skip_ref[...])


def _combine(agg, dis_col, skip):
    grid = (NPAD // BLK,)
    return pl.pallas_call(
        _comb_body,
        grid=grid,
        in_specs=[
            pl.BlockSpec((NC, BLK, D), lambda i: (0, i, 0)),
            pl.BlockSpec((BLK, 1), lambda i: (i, 0)),
            pl.BlockSpec((BLK, D), lambda i: (i, 0)),
        ],
        out_specs=pl.BlockSpec((BLK, D), lambda i: (i, 0)),
        out_shape=jax.ShapeDtypeStruct((NPAD, D), jnp.float32),
    )(agg, dis_col, skip)


def kernel(x, edge_index, skip_w, skip_b, msg_w, msg_b):
    x = x.astype(jnp.float32)
    ei = edge_index.astype(jnp.int32)

    x_pad = jnp.zeros((NPAD, D), jnp.float32).at[:N].set(x)

    pad = jnp.full((2, EPAD - E), NPAD - 1, jnp.int32)
    ei_p = jnp.concatenate([ei, pad], axis=1)
    src = ei_p[0].reshape(NW, GSTEPS, B)
    dst = ei_p[1].reshape(NW, GSTEPS, B)
    srcg = jnp.stack([src, src + NPAD], axis=0)

    y, skip = _pre(x_pad, msg_w, skip_w,
                   skip_b.reshape(1, D), msg_b.reshape(1, D))
    dis, _feat, agg = _sc_agg(src, srcg, dst, y)
    out = _combine(agg, dis[0].reshape(NPAD, 1), skip)
    return out[:N]
